# baseline (speedup 1.0000x reference)
import functools

import jax
import jax.numpy as jnp
from jax import lax
from jax.experimental import pallas as pl
from jax.experimental.pallas import tpu as pltpu
from jax.experimental.pallas import tpu_sc as plsc

_VOCAB = 1000000
_D = 64
_B = 4096
_H = 50
_N = _B * _H
_SCALE = 8.0

_NC = 2
_NS = 16
_NW = _NC * _NS
_C = 128
_CHUNKS = _N // (_NW * _C)

_FULLCOLS = _VOCAB // _C
_TAIL = _VOCAB - _FULLCOLS * _C
_PERW = _FULLCOLS // _NW
_EXTRA = _FULLCOLS - _PERW * _NW

_mesh = plsc.VectorSubcoreMesh(
    core_axis_name="c", subcore_axis_name="s",
    num_cores=_NC, num_subcores=_NS)


_BLK = _C * 2 * _D


@functools.partial(
    pl.kernel,
    out_type=jax.ShapeDtypeStruct((_VOCAB * 2 * _D,), jnp.float32),
    mesh=_mesh,
    scratch_types=[
        pltpu.VMEM((_D, _C + 1), jnp.float32),
        pltpu.VMEM((_D, _C + 1), jnp.float32),
        pltpu.VMEM((_BLK,), jnp.float32),
        pltpu.VMEM((_BLK,), jnp.float32),
        pltpu.SemaphoreType.DMA,
        pltpu.SemaphoreType.DMA,
        pltpu.SemaphoreType.DMA,
        pltpu.SemaphoreType.DMA,
    ],
    compiler_params=pltpu.CompilerParams(needs_layout_passes=False),
)
def _fmt(tabt_hbm, tailp_hbm, out_hbm, tin0, tin1, tout0, tout1,
         isem0, isem1, osem0, osem1):
    tins = (tin0, tin1)
    touts = (tout0, tout1)
    isems = (isem0, isem1)
    osems = (osem0, osem1)
    wid = lax.axis_index("s") * _NC + lax.axis_index("c")
    base = wid * _PERW + jnp.minimum(wid, _EXTRA)
    count = jnp.where(wid < _EXTRA, _PERW + 1, _PERW)
    iota = lax.iota(jnp.int32, 16)

    def col_of(i):
        return base + i

    def start_in(i, b):
        cb = col_of(i)
        for c in range(_D):
            pltpu.async_copy(
                tabt_hbm.at[c, pl.ds(cb * _C, _C)],
                tins[b].at[c, pl.ds(0, _C)], isems[b])

    def wait_in(i, b):
        cb = col_of(i)
        for c in range(_D):
            pltpu.make_async_copy(
                tabt_hbm.at[c, pl.ds(cb * _C, _C)],
                tins[b].at[c, pl.ds(0, _C)], isems[b]).wait()

    def wb(i, b):
        cb = col_of(i)
        return out_hbm.at[pl.ds(cb * _BLK, _BLK)]

    def transpose(b):
        tin = tins[b]
        tout = touts[b]
        for r in range(_C):
            for g in range(_D // 16):
                vals = plsc.load_gather(tin, [iota + 16 * g,
                                              jnp.full((16,), r, jnp.int32)])
                tout[pl.ds(r * 2 * _D + 16 * g, 16)] = vals

    @pl.when(count > 0)
    def _():
        start_in(0, 0)

    @pl.when(count > 1)
    def _():
        start_in(1, 1)

    @pl.loop(0, (_PERW + 2) // 2)
    def _(g2):
        for b in (0, 1):
            i = 2 * g2 + b

            @pl.when(i < count)
            def _():
                wait_in(i, b)

                @pl.when(i >= 2)
                def _():
                    pltpu.make_async_copy(touts[b], wb(i - 2, b),
                                          osems[b]).wait()

                transpose(b)
                pltpu.async_copy(touts[b], wb(i, b), osems[b])

                @pl.when(i + 2 < count)
                def _():
                    start_in(i + 2, b)

    @pl.when(count >= 2)
    def _():
        pltpu.make_async_copy(
            tout0, wb(count - 2, 0), osems[0]).wait()

    @pl.when(count >= 1)
    def _():
        pltpu.make_async_copy(
            tout1, wb(count - 1, 1), osems[1]).wait()

    @pl.when(wid == _NW - 1)
    def _():
        tail_f = _TAIL * 2 * _D
        pltpu.sync_copy(tailp_hbm, tout0.at[pl.ds(0, tail_f)])
        pltpu.sync_copy(tout0.at[pl.ds(0, tail_f)],
                        out_hbm.at[pl.ds(_FULLCOLS * _BLK, tail_f)])


@functools.partial(
    pl.kernel,
    out_type=jax.ShapeDtypeStruct((_N, _D), jnp.float32),
    mesh=_mesh,
    scratch_types=[
        pltpu.VMEM((_CHUNKS, _C), jnp.int32),
        pltpu.VMEM((2, _C, 2 * _D), jnp.float32),
        pltpu.VMEM((2, _C, _D), jnp.float32),
        pltpu.SemaphoreType.DMA,
        pltpu.SemaphoreType.DMA,
        pltpu.SemaphoreType.DMA,
        pltpu.SemaphoreType.DMA,
    ],
    compiler_params=pltpu.CompilerParams(use_tc_tiling_on_sc=False),
)
def _lookup(idx_hbm, tab_hbm, out_hbm, idx_v, rows_v, out_v,
            gsem0, gsem1, osem0, osem1):
    gsems = (gsem0, gsem1)
    osems = (osem0, osem1)
    wid = lax.axis_index("s") * _NC + lax.axis_index("c")
    pltpu.sync_copy(idx_hbm.at[wid], idx_v)

    rbase0 = pl.multiple_of(wid * (_CHUNKS * _C), _C)

    def start_gather(ck, b):
        pltpu.async_copy(tab_hbm.at[idx_v.at[ck]], rows_v.at[b], gsems[b])

    def wait_gather(ck, b):
        pltpu.make_async_copy(
            tab_hbm.at[idx_v.at[ck]], rows_v.at[b], gsems[b]).wait()

    def wb_slice(ck):
        rb = pl.multiple_of(rbase0 + ck * _C, _C)
        return out_hbm.at[pl.ds(rb, _C)]

    def extract(b):
        rows = rows_v.at[b]
        outb = out_v.at[b]

        @plsc.parallel_loop(0, _C, 1, unroll=4)
        def _(j):
            for g in range(_D // 16):
                sl = pl.ds(16 * g, 16)
                outb[j, sl] = rows[j, sl] * _SCALE

    start_gather(0, 0)
    start_gather(1, 1)

    @pl.loop(0, _CHUNKS // 2)
    def _(g2):
        for b in (0, 1):
            ck = 2 * g2 + b
            wait_gather(ck, b)

            @pl.when(ck >= 2)
            def _():
                pltpu.make_async_copy(
                    out_v.at[b], wb_slice(ck - 2), osems[b]).wait()

            extract(b)
            pltpu.async_copy(out_v.at[b], wb_slice(ck), osems[b])

            @pl.when(ck + 2 < _CHUNKS)
            def _():
                start_gather(ck + 2, b)

    pltpu.make_async_copy(out_v.at[0], wb_slice(_CHUNKS - 2), osems[0]).wait()
    pltpu.make_async_copy(out_v.at[1], wb_slice(_CHUNKS - 1), osems[1]).wait()


@jax.jit
def _sc_embed(idx3, tabt, tailp):
    tabp = _fmt(tabt, tailp)
    return _lookup(idx3, tabp.reshape(_VOCAB, 2 * _D))


def kernel(inputs, embeddings):
    idx3 = inputs.astype(jnp.int32).reshape(_N).reshape(_NW, _CHUNKS, _C)
    tailp = jnp.pad(embeddings[_FULLCOLS * _C:], ((0, 0), (0, _D)))
    out = _sc_embed(idx3, embeddings.T, tailp.reshape(_TAIL * 2 * _D))
    return out.reshape(_B, _H, _D)


# speedup vs baseline: 3.0294x; 3.0294x over previous
import functools

import jax
import jax.numpy as jnp
from jax import lax
from jax.experimental import pallas as pl
from jax.experimental.pallas import tpu as pltpu
from jax.experimental.pallas import tpu_sc as plsc

_VOCAB = 1000000
_D = 64
_B = 4096
_H = 50
_N = _B * _H
_SCALE = 8.0

_NC = 2
_NS = 16
_NW = _NC * _NS
_C = 128
_CHUNKS = _N // (_NW * _C)


@jax.jit
def _sc_embed(idx3, tabp):
    mesh = plsc.VectorSubcoreMesh(
        core_axis_name="c", subcore_axis_name="s",
        num_cores=_NC, num_subcores=_NS)

    @functools.partial(
        pl.kernel,
        out_type=jax.ShapeDtypeStruct((_N, _D), jnp.float32),
        mesh=mesh,
        scratch_types=[
            pltpu.VMEM((_CHUNKS, _C), jnp.int32),
            pltpu.VMEM((2, _C, 2 * _D), jnp.float32),
            pltpu.VMEM((2, _C, _D), jnp.float32),
            pltpu.SemaphoreType.DMA,
            pltpu.SemaphoreType.DMA,
            pltpu.SemaphoreType.DMA,
            pltpu.SemaphoreType.DMA,
        ],
        compiler_params=pltpu.CompilerParams(use_tc_tiling_on_sc=False),
    )
    def k(idx_hbm, tab_hbm, out_hbm, idx_v, rows_v, out_v,
          gsem0, gsem1, osem0, osem1):
        gsems = (gsem0, gsem1)
        osems = (osem0, osem1)
        wid = lax.axis_index("s") * _NC + lax.axis_index("c")
        pltpu.sync_copy(idx_hbm.at[wid], idx_v)

        rbase0 = pl.multiple_of(wid * (_CHUNKS * _C), _C)

        def start_gather(ck, b):
            pltpu.async_copy(tab_hbm.at[idx_v.at[ck]], rows_v.at[b], gsems[b])

        def wait_gather(ck, b):
            pltpu.make_async_copy(
                tab_hbm.at[idx_v.at[ck]], rows_v.at[b], gsems[b]).wait()

        def wb_slice(ck):
            rb = pl.multiple_of(rbase0 + ck * _C, _C)
            return out_hbm.at[pl.ds(rb, _C)]

        def extract(b):
            rows = rows_v.at[b]
            outb = out_v.at[b]

            @plsc.parallel_loop(0, _C, 1, unroll=4)
            def _(j):
                for g in range(_D // 16):
                    sl = pl.ds(16 * g, 16)
                    outb[j, sl] = rows[j, sl] * _SCALE

        start_gather(0, 0)
        start_gather(1, 1)

        @pl.loop(0, _CHUNKS // 2)
        def _(g2):
            for b in (0, 1):
                ck = 2 * g2 + b
                wait_gather(ck, b)

                @pl.when(ck >= 2)
                def _():
                    pltpu.make_async_copy(
                        out_v.at[b], wb_slice(ck - 2), osems[b]).wait()

                extract(b)
                pltpu.async_copy(out_v.at[b], wb_slice(ck), osems[b])

                @pl.when(ck + 2 < _CHUNKS)
                def _():
                    start_gather(ck + 2, b)

        pltpu.make_async_copy(
            out_v.at[0], wb_slice(_CHUNKS - 2), osems[0]).wait()
        pltpu.make_async_copy(
            out_v.at[1], wb_slice(_CHUNKS - 1), osems[1]).wait()

    return k(idx3, tabp)


def kernel(inputs, embeddings):
    idx3 = inputs.astype(jnp.int32).reshape(_N).reshape(_NW, _CHUNKS, _C)
    tabp = jnp.pad(embeddings, ((0, 0), (0, _D)))
    out = _sc_embed(idx3, tabp)
    return out.reshape(_B, _H, _D)
